# baseline (speedup 1.0000x reference)
import functools

import jax
import jax.numpy as jnp
from jax import lax
from jax.experimental import pallas as pl
from jax.experimental.pallas import tpu as pltpu
from jax.experimental.pallas import tpu_sc as plsc

DIM = 768
LANES = 16
KCHUNKS = DIM // LANES
EPS = 1e-12

NC = 2
NS = 16
NW = NC * NS

CHUNK = 32
XRING = 3
PRING = 2


def _lane_sum_splat(v):
    idx = lax.iota(jnp.int32, LANES)
    for sh in (8, 4, 2, 1):
        perm = jnp.bitwise_xor(idx, sh)
        v = v + v.at[perm].get(mode="promise_in_bounds")
    return v


def _rsqrt_vec(x):
    i = lax.bitcast_convert_type(x, jnp.int32)
    i = jnp.int32(0x5F3759DF) - lax.shift_right_logical(i, 1)
    y = lax.bitcast_convert_type(i, jnp.float32)
    for _ in range(3):
        y = y * (1.5 - 0.5 * x * y * y)
    return y


def _make_sc_call(total_rows, seq_len):
    rows_per_w = total_rows // NW
    n_chunks = rows_per_w // CHUNK
    batch = total_rows // seq_len
    ppw = seq_len // NW
    npc = ppw // CHUNK
    mesh = plsc.VectorSubcoreMesh(
        core_axis_name="c", subcore_axis_name="s",
        num_cores=NC, num_subcores=NS)

    @functools.partial(
        pl.kernel,
        out_type=jax.ShapeDtypeStruct((total_rows, DIM), jnp.float32),
        mesh=mesh,
        scratch_types=[
            pltpu.VMEM((rows_per_w,), jnp.int32),
            pltpu.VMEM((XRING, CHUNK, DIM), jnp.float32),
            pltpu.VMEM((PRING, CHUNK, DIM), jnp.float32),
            pltpu.VMEM((DIM,), jnp.float32),
            pltpu.VMEM((DIM,), jnp.float32),
            pltpu.VMEM((LANES,), jnp.int32),
            pltpu.SemaphoreType.DMA,
            pltpu.SemaphoreType.DMA,
            pltpu.SemaphoreType.DMA,
        ],
    )
    def sc_embed(ids_hbm, tok_hbm, pos_hbm, gamma_hbm, beta_hbm, unif_hbm,
                 out_hbm,
                 idx_v, xb, pb, gamma_v, beta_v, unif_s, gsem, psem, osem):
        wid = lax.axis_index("s") * NC + lax.axis_index("c")
        pbase = wid * ppw

        pltpu.sync_copy(gamma_hbm, gamma_v)
        pltpu.sync_copy(beta_hbm, beta_v)
        pltpu.sync_copy(unif_hbm, unif_s)
        for b in range(batch):
            pltpu.sync_copy(
                ids_hbm.at[pl.ds(b * seq_len + pbase, ppw)],
                idx_v.at[pl.ds(b * ppw, ppw)])
        unif = unif_s[pl.ds(0, LANES)][0]

        def _split(c):
            return lax.div(c, batch), lax.rem(c, batch)

        def issue_gather(c):
            bx = lax.rem(c, XRING)
            pc, b = _split(c)
            pltpu.async_copy(
                tok_hbm.at[idx_v.at[pl.ds(b * ppw + pc * CHUNK, CHUNK)]],
                xb.at[bx], gsem)

        def issue_pos(pc):
            px = lax.rem(pc, PRING)
            pltpu.async_copy(pos_hbm.at[pl.ds(pbase + pc * CHUNK, CHUNK), :],
                             pb.at[px], psem)

        def wait_gather():
            pltpu.make_async_copy(
                tok_hbm.at[idx_v.at[pl.ds(0, CHUNK)]], xb.at[0], gsem).wait()

        def wait_pos():
            pltpu.make_async_copy(pos_hbm.at[pl.ds(0, CHUNK), :],
                                  pb.at[0], psem).wait()

        def wait_out():
            pltpu.make_async_copy(xb.at[0],
                                  out_hbm.at[pl.ds(0, CHUNK), :], osem).wait()

        issue_gather(0)
        issue_pos(0)

        def step(s, _):
            nxt = s + 1
            pc, b = _split(s)

            @pl.when(jnp.logical_and(nxt < n_chunks, s >= XRING - 1))
            def _():
                wait_out()

            @pl.when(nxt < n_chunks)
            def _():
                issue_gather(nxt)

            @pl.when(jnp.logical_and(b == 0, pc + 1 < npc))
            def _():
                issue_pos(pc + 1)

            wait_gather()

            @pl.when(b == 0)
            def _():
                wait_pos()

            bx = lax.rem(s, XRING)
            px = lax.rem(pc, PRING)

            def token_body(j, apply_gb):
                zero = jnp.zeros((LANES,), jnp.float32)
                acc_s, acc_q = zero, zero
                xs = []
                for k in range(KCHUNKS):
                    sl = pl.ds(k * LANES, LANES)
                    v = xb[bx, j, sl] + pb[px, j, sl]
                    xs.append(v)
                    acc_s = acc_s + v
                    acc_q = acc_q + v * v

                s_vec = _lane_sum_splat(acc_s) * (1.0 / DIM)
                q_vec = _lane_sum_splat(acc_q) * (1.0 / DIM)
                var_vec = q_vec - s_vec * s_vec
                r_vec = _rsqrt_vec(var_vec + EPS)

                for k in range(KCHUNKS):
                    sl = pl.ds(k * LANES, LANES)
                    y = (xs[k] - s_vec) * r_vec
                    if apply_gb:
                        y = y * gamma_v[sl] + beta_v[sl]
                    xb[bx, j, sl] = y

            @pl.when(unif != 0)
            def _():
                plsc.parallel_loop(0, CHUNK, unroll=2)(
                    lambda j: token_body(j, False))

            @pl.when(unif == 0)
            def _():
                plsc.parallel_loop(0, CHUNK, unroll=1)(
                    lambda j: token_body(j, True))

            pltpu.async_copy(
                xb.at[bx],
                out_hbm.at[pl.ds(b * seq_len + pbase + pc * CHUNK, CHUNK), :],
                osem)
            return 0

        lax.fori_loop(0, n_chunks, step, 0)

        for _ in range(min(XRING, n_chunks)):
            wait_out()

    return sc_embed


def kernel(input_ids, token_table, pos_table, ln_gamma, ln_beta):
    batch, seq_len = input_ids.shape
    total_rows = batch * seq_len
    ids_flat = input_ids.reshape(total_rows).astype(jnp.int32)
    unif = (jnp.all(ln_gamma == 1.0) & jnp.all(ln_beta == 0.0))
    unif = jnp.full((LANES,), unif.astype(jnp.int32))
    sc_call = _make_sc_call(total_rows, seq_len)
    out = sc_call(ids_flat, token_table, pos_table, ln_gamma, ln_beta, unif)
    return out.reshape(batch, seq_len, DIM)


# speedup vs baseline: 1.0239x; 1.0239x over previous
import functools

import jax
import jax.numpy as jnp
from jax import lax
from jax.experimental import pallas as pl
from jax.experimental.pallas import tpu as pltpu
from jax.experimental.pallas import tpu_sc as plsc

DIM = 768
LANES = 16
KCHUNKS = DIM // LANES
EPS = 1e-12

NC = 2
NS = 16
NW = NC * NS

CHUNK = 32
XRING = 3
PRING = 2


def _lane_sum_splat(v):
    idx = lax.iota(jnp.int32, LANES)
    for sh in (8, 4, 2, 1):
        perm = jnp.bitwise_xor(idx, sh)
        v = v + v.at[perm].get(mode="promise_in_bounds")
    return v


def _rsqrt_vec(x):
    i = lax.bitcast_convert_type(x, jnp.int32)
    i = jnp.int32(0x5F3759DF) - lax.shift_right_logical(i, 1)
    y = lax.bitcast_convert_type(i, jnp.float32)
    for _ in range(3):
        y = y * (1.5 - 0.5 * x * y * y)
    return y


def _make_sc_call(total_rows, seq_len):
    rows_per_w = total_rows // NW
    n_chunks = rows_per_w // CHUNK
    batch = total_rows // seq_len
    ppw = seq_len // NW
    npc = ppw // CHUNK
    mesh = plsc.VectorSubcoreMesh(
        core_axis_name="c", subcore_axis_name="s",
        num_cores=NC, num_subcores=NS)

    @functools.partial(
        pl.kernel,
        out_type=jax.ShapeDtypeStruct((total_rows, DIM), jnp.float32),
        mesh=mesh,
        scratch_types=[
            pltpu.VMEM((rows_per_w,), jnp.int32),
            pltpu.VMEM((XRING, CHUNK, DIM), jnp.float32),
            pltpu.VMEM((PRING, CHUNK, DIM), jnp.float32),
            pltpu.VMEM((DIM,), jnp.float32),
            pltpu.VMEM((DIM,), jnp.float32),
            pltpu.VMEM((LANES,), jnp.int32),
            pltpu.SemaphoreType.DMA,
            pltpu.SemaphoreType.DMA,
            pltpu.SemaphoreType.DMA,
        ],
    )
    def sc_embed(ids_hbm, tok_hbm, pos_hbm, gamma_hbm, beta_hbm, unif_hbm,
                 out_hbm,
                 idx_v, xb, pb, gamma_v, beta_v, unif_s, gsem, psem, osem):
        wid = lax.axis_index("s") * NC + lax.axis_index("c")
        pbase = wid * ppw

        pltpu.sync_copy(gamma_hbm, gamma_v)
        pltpu.sync_copy(beta_hbm, beta_v)
        pltpu.sync_copy(unif_hbm, unif_s)
        for b in range(batch):
            pltpu.sync_copy(
                ids_hbm.at[pl.ds(b * seq_len + pbase, ppw)],
                idx_v.at[pl.ds(b * ppw, ppw)])
        unif = unif_s[pl.ds(0, LANES)][0]

        def _split(c):
            return lax.div(c, batch), lax.rem(c, batch)

        def issue_gather(c):
            bx = lax.rem(c, XRING)
            pc, b = _split(c)
            pltpu.async_copy(
                tok_hbm.at[idx_v.at[pl.ds(b * ppw + pc * CHUNK, CHUNK)]],
                xb.at[bx], gsem)

        def issue_pos(pc):
            px = lax.rem(pc, PRING)
            pltpu.async_copy(pos_hbm.at[pl.ds(pbase + pc * CHUNK, CHUNK), :],
                             pb.at[px], psem)

        def wait_gather():
            pltpu.make_async_copy(
                tok_hbm.at[idx_v.at[pl.ds(0, CHUNK)]], xb.at[0], gsem).wait()

        def wait_pos():
            pltpu.make_async_copy(pos_hbm.at[pl.ds(0, CHUNK), :],
                                  pb.at[0], psem).wait()

        def wait_out():
            pltpu.make_async_copy(xb.at[0],
                                  out_hbm.at[pl.ds(0, CHUNK), :], osem).wait()

        issue_gather(0)
        issue_pos(0)

        def step(s, _):
            nxt = s + 1
            pc, b = _split(s)

            @pl.when(jnp.logical_and(nxt < n_chunks, s >= XRING - 1))
            def _():
                wait_out()

            @pl.when(nxt < n_chunks)
            def _():
                issue_gather(nxt)

            @pl.when(jnp.logical_and(b == 0, pc + 1 < npc))
            def _():
                issue_pos(pc + 1)

            wait_gather()

            @pl.when(b == 0)
            def _():
                wait_pos()

            bx = lax.rem(s, XRING)
            px = lax.rem(pc, PRING)

            def token_body(j, apply_gb):
                zero = jnp.zeros((LANES,), jnp.float32)
                acc_s, acc_q = zero, zero
                xs = []
                for k in range(KCHUNKS):
                    sl = pl.ds(k * LANES, LANES)
                    v = xb[bx, j, sl] + pb[px, j, sl]
                    xs.append(v)
                    acc_s = acc_s + v
                    acc_q = acc_q + v * v

                s_vec = _lane_sum_splat(acc_s) * (1.0 / DIM)
                q_vec = _lane_sum_splat(acc_q) * (1.0 / DIM)
                var_vec = q_vec - s_vec * s_vec
                r_vec = _rsqrt_vec(var_vec + EPS)

                for k in range(KCHUNKS):
                    sl = pl.ds(k * LANES, LANES)
                    y = (xs[k] - s_vec) * r_vec
                    if apply_gb:
                        y = y * gamma_v[sl] + beta_v[sl]
                    xb[bx, j, sl] = y

            @pl.when(unif != 0)
            def _():
                plsc.parallel_loop(0, CHUNK, unroll=1)(
                    lambda j: token_body(j, False))

            @pl.when(unif == 0)
            def _():
                plsc.parallel_loop(0, CHUNK, unroll=1)(
                    lambda j: token_body(j, True))

            pltpu.async_copy(
                xb.at[bx],
                out_hbm.at[pl.ds(b * seq_len + pbase + pc * CHUNK, CHUNK), :],
                osem)
            return 0

        lax.fori_loop(0, n_chunks, step, 0)

        for _ in range(min(XRING, n_chunks)):
            wait_out()

    return sc_embed


def kernel(input_ids, token_table, pos_table, ln_gamma, ln_beta):
    batch, seq_len = input_ids.shape
    total_rows = batch * seq_len
    ids_flat = input_ids.reshape(total_rows).astype(jnp.int32)
    unif = (jnp.all(ln_gamma == 1.0) & jnp.all(ln_beta == 0.0))
    unif = jnp.full((LANES,), unif.astype(jnp.int32))
    sc_call = _make_sc_call(total_rows, seq_len)
    out = sc_call(ids_flat, token_table, pos_table, ln_gamma, ln_beta, unif)
    return out.reshape(batch, seq_len, DIM)
